# baseline (speedup 1.0000x reference)
import functools

import jax
import jax.numpy as jnp
from jax import lax
from jax.experimental import pallas as pl
from jax.experimental.pallas import tpu as pltpu
from jax.experimental.pallas import tpu_sc as plsc

NC, NS, L = 2, 16, 16
NW = NC * NS
B = 16384
BPW = B // NW
NV = BPW // L
HALF = BPW // 2
U_ROWS, I_ROWS, D = 1500, 2000, 3

_mesh = plsc.VectorSubcoreMesh(core_axis_name="c", subcore_axis_name="s")


@functools.partial(
    pl.kernel,
    out_type=jax.ShapeDtypeStruct((B,), jnp.float32),
    mesh=_mesh,
    scratch_types=[
        pltpu.VMEM((2, BPW), jnp.int32),
        pltpu.VMEM((U_ROWS * D,), jnp.float32),
        pltpu.VMEM((I_ROWS * D,), jnp.float32),
        pltpu.VMEM((BPW,), jnp.float32),
        pltpu.SemaphoreType.DMA,
        pltpu.SemaphoreType.DMA,
    ],
    compiler_params=pltpu.CompilerParams(
        needs_layout_passes=False, skip_device_barrier=True,
        use_tc_tiling_on_sc=False),
)
def _mf_kernel(data_hbm, utab_hbm, itab_hbm, out_hbm,
               idx_v, utab_v, itab_v, out_v, sem, osem):
    wid = lax.axis_index("s") * NC + lax.axis_index("c")
    base = wid * BPW

    c1 = pltpu.make_async_copy(data_hbm.at[:, pl.ds(base, BPW)], idx_v, sem)
    c2 = pltpu.make_async_copy(utab_hbm, utab_v, sem)
    c3 = pltpu.make_async_copy(itab_hbm, itab_v, sem)
    c1.start()
    c2.start()
    c3.start()
    c1.wait()
    c2.wait()
    c3.wait()

    three = jnp.full((L,), D, jnp.int32)
    out_half = [
        pltpu.make_async_copy(out_v.at[pl.ds(h * HALF, HALF)],
                              out_hbm.at[pl.ds(base + h * HALF, HALF)], osem)
        for h in range(2)
    ]
    for i in range(NV):
        ub = idx_v[0, pl.ds(i * L, L)] * three
        vb = idx_v[1, pl.ds(i * L, L)] * three
        acc = None
        for d in range(D):
            uu = plsc.load_gather(utab_v, [ub + d])
            vv = plsc.load_gather(itab_v, [vb + d])
            prod = uu * vv
            acc = prod if acc is None else acc + prod
        out_v[pl.ds(i * L, L)] = acc
        if i == NV // 2 - 1:
            out_half[0].start()
    out_half[1].start()
    out_half[0].wait()
    out_half[1].wait()


def kernel(data, user_factors, item_factors):
    data = data.astype(jnp.int32)
    return _mf_kernel(data,
                      user_factors.reshape(-1), item_factors.reshape(-1))


# speedup vs baseline: 1.1124x; 1.1124x over previous
import functools

import jax
import jax.numpy as jnp
from jax import lax
from jax.experimental import pallas as pl
from jax.experimental.pallas import tpu as pltpu
from jax.experimental.pallas import tpu_sc as plsc

NC, NS, L = 1, 16, 16
NW = NC * NS
B = 16384
BPW = B // NW
NV = BPW // L
HALF = BPW // 2
U_ROWS, I_ROWS, D = 1500, 2000, 3

_mesh = plsc.VectorSubcoreMesh(core_axis_name="c", subcore_axis_name="s", num_cores=1)


@functools.partial(
    pl.kernel,
    out_type=jax.ShapeDtypeStruct((B,), jnp.float32),
    mesh=_mesh,
    scratch_types=[
        pltpu.VMEM((2, BPW), jnp.int32),
        pltpu.VMEM((U_ROWS * D,), jnp.float32),
        pltpu.VMEM((I_ROWS * D,), jnp.float32),
        pltpu.VMEM((BPW,), jnp.float32),
        pltpu.SemaphoreType.DMA,
        pltpu.SemaphoreType.DMA,
    ],
    compiler_params=pltpu.CompilerParams(
        needs_layout_passes=False, skip_device_barrier=True),
)
def _mf_kernel(data_hbm, utab_hbm, itab_hbm, out_hbm,
               idx_v, utab_v, itab_v, out_v, sem, osem):
    wid = lax.axis_index("s") * NC + lax.axis_index("c")
    base = wid * BPW

    c1 = pltpu.make_async_copy(data_hbm.at[:, pl.ds(base, BPW)], idx_v, sem)
    c2 = pltpu.make_async_copy(utab_hbm, utab_v, sem)
    c3 = pltpu.make_async_copy(itab_hbm, itab_v, sem)
    c1.start()
    c2.start()
    c3.start()
    c1.wait()
    c2.wait()
    c3.wait()

    three = jnp.full((L,), D, jnp.int32)
    out_half = [
        pltpu.make_async_copy(out_v.at[pl.ds(h * HALF, HALF)],
                              out_hbm.at[pl.ds(base + h * HALF, HALF)], osem)
        for h in range(2)
    ]
    for i in range(NV):
        ub = idx_v[0, pl.ds(i * L, L)] * three
        vb = idx_v[1, pl.ds(i * L, L)] * three
        acc = None
        for d in range(D):
            uu = plsc.load_gather(utab_v, [ub + d])
            vv = plsc.load_gather(itab_v, [vb + d])
            prod = uu * vv
            acc = prod if acc is None else acc + prod
        out_v[pl.ds(i * L, L)] = acc
        if i == NV // 2 - 1:
            out_half[0].start()
    out_half[1].start()
    out_half[0].wait()
    out_half[1].wait()


def kernel(data, user_factors, item_factors):
    data = data.astype(jnp.int32)
    return _mf_kernel(data,
                      user_factors.reshape(-1), item_factors.reshape(-1))


# speedup vs baseline: 1.1124x; 1.0000x over previous
import functools

import jax
import jax.numpy as jnp
from jax import lax
from jax.experimental import pallas as pl
from jax.experimental.pallas import tpu as pltpu
from jax.experimental.pallas import tpu_sc as plsc

NC, NS, L = 1, 16, 16
NW = NC * NS
B = 16384
BPW = B // NW
NV = BPW // L
HALF = BPW // 2
U_ROWS, I_ROWS, D = 1500, 2000, 3

_mesh = plsc.VectorSubcoreMesh(core_axis_name="c", subcore_axis_name="s", num_cores=1)


@functools.partial(
    pl.kernel,
    out_type=jax.ShapeDtypeStruct((B,), jnp.float32),
    mesh=_mesh,
    scratch_types=[
        pltpu.VMEM((2, BPW), jnp.int32),
        pltpu.VMEM((U_ROWS * D,), jnp.float32),
        pltpu.VMEM((I_ROWS * D,), jnp.float32),
        pltpu.VMEM((BPW,), jnp.float32),
        pltpu.SemaphoreType.DMA,
        pltpu.SemaphoreType.DMA,
    ],
    compiler_params=pltpu.CompilerParams(
        needs_layout_passes=False, skip_device_barrier=True,
        disable_bounds_checks=True, disable_semaphore_checks=True),
)
def _mf_kernel(data_hbm, utab_hbm, itab_hbm, out_hbm,
               idx_v, utab_v, itab_v, out_v, sem, osem):
    wid = lax.axis_index("s") * NC + lax.axis_index("c")
    base = wid * BPW

    c1 = pltpu.make_async_copy(data_hbm.at[:, pl.ds(base, BPW)], idx_v, sem)
    c2 = pltpu.make_async_copy(utab_hbm, utab_v, sem)
    c3 = pltpu.make_async_copy(itab_hbm, itab_v, sem)
    c1.start()
    c2.start()
    c3.start()
    c1.wait()
    c2.wait()
    c3.wait()

    three = jnp.full((L,), D, jnp.int32)
    out_half = [
        pltpu.make_async_copy(out_v.at[pl.ds(h * HALF, HALF)],
                              out_hbm.at[pl.ds(base + h * HALF, HALF)], osem)
        for h in range(2)
    ]
    for i in range(NV):
        ub = idx_v[0, pl.ds(i * L, L)] * three
        vb = idx_v[1, pl.ds(i * L, L)] * three
        acc = None
        for d in range(D):
            uu = plsc.load_gather(utab_v, [ub + d])
            vv = plsc.load_gather(itab_v, [vb + d])
            prod = uu * vv
            acc = prod if acc is None else acc + prod
        out_v[pl.ds(i * L, L)] = acc
        if i == NV // 2 - 1:
            out_half[0].start()
    out_half[1].start()
    out_half[0].wait()
    out_half[1].wait()


def kernel(data, user_factors, item_factors):
    data = data.astype(jnp.int32)
    return _mf_kernel(data,
                      user_factors.reshape(-1), item_factors.reshape(-1))


# speedup vs baseline: 1.1635x; 1.0460x over previous
import functools

import jax
import jax.numpy as jnp
from jax import lax
from jax.experimental import pallas as pl
from jax.experimental.pallas import tpu as pltpu
from jax.experimental.pallas import tpu_sc as plsc

NC, NS, L = 1, 16, 16
NW = NC * NS
B = 16384
BPW = B // NW
NV = BPW // L
HALF = BPW // 2
U_ROWS, I_ROWS, D = 1500, 2000, 3

_mesh = plsc.VectorSubcoreMesh(core_axis_name="c", subcore_axis_name="s", num_cores=1)


@functools.partial(
    pl.kernel,
    out_type=jax.ShapeDtypeStruct((B,), jnp.float32),
    mesh=_mesh,
    scratch_types=[
        pltpu.VMEM((2, BPW), jnp.int32),
        pltpu.VMEM((U_ROWS * D,), jnp.float32),
        pltpu.VMEM((I_ROWS * D,), jnp.float32),
        pltpu.VMEM((BPW,), jnp.float32),
        pltpu.SemaphoreType.DMA,
        pltpu.SemaphoreType.DMA,
    ],
    compiler_params=pltpu.CompilerParams(
        needs_layout_passes=False, skip_device_barrier=True),
)
def _mf_kernel(data_hbm, utab_hbm, itab_hbm, out_hbm,
               idx_v, utab_v, itab_v, out_v, sem, osem):
    wid = lax.axis_index("s") * NC + lax.axis_index("c")
    base = wid * BPW

    c1 = pltpu.make_async_copy(data_hbm.at[:, pl.ds(base, BPW)], idx_v, sem)
    c2 = pltpu.make_async_copy(utab_hbm, utab_v, sem)
    c3 = pltpu.make_async_copy(itab_hbm, itab_v, sem)
    c1.start()
    c2.start()
    c3.start()
    c1.wait()
    c2.wait()
    c3.wait()

    three = jnp.full((L,), D, jnp.int32)
    out_half = [
        pltpu.make_async_copy(out_v.at[pl.ds(h * HALF, HALF)],
                              out_hbm.at[pl.ds(base + h * HALF, HALF)], osem)
        for h in range(2)
    ]

    def body(i):
        off = i * L
        ub = idx_v[0, pl.ds(off, L)] * three
        vb = idx_v[1, pl.ds(off, L)] * three
        acc = None
        for d in range(D):
            uu = plsc.load_gather(utab_v, [ub + d])
            vv = plsc.load_gather(itab_v, [vb + d])
            prod = uu * vv
            acc = prod if acc is None else acc + prod
        out_v[pl.ds(off, L)] = acc

    plsc.parallel_loop(0, NV // 2, unroll=4)(body)
    out_half[0].start()
    plsc.parallel_loop(NV // 2, NV, unroll=4)(body)
    out_half[1].start()
    out_half[0].wait()
    out_half[1].wait()


def kernel(data, user_factors, item_factors):
    data = data.astype(jnp.int32)
    return _mf_kernel(data,
                      user_factors.reshape(-1), item_factors.reshape(-1))
